# baseline (speedup 1.0000x reference)
import functools
import jax
import jax.numpy as jnp
from jax import lax
from jax.experimental import pallas as pl
from jax.experimental.pallas import tpu as pltpu
from jax.experimental.pallas import tpu_sc as plsc

N = 4096
E = 65536
D_IN = 128
CH = 32
OUT = 16
NB = 64
K1 = 2048
K2 = 1024

_HI = lax.Precision.HIGHEST


def _mm_nn_kernel(prec, a_ref, b_ref, o_ref):
    o_ref[...] = jnp.dot(a_ref[...], b_ref[...], precision=prec,
                         preferred_element_type=jnp.float32)


def mm_nn(a, b, bm=512):
    prec = _HI if a.dtype == jnp.float32 else None
    M, K = a.shape
    _, Nn = b.shape
    bm = min(bm, M)
    return pl.pallas_call(
        functools.partial(_mm_nn_kernel, prec),
        grid=(M // bm,),
        in_specs=[pl.BlockSpec((bm, K), lambda i: (i, 0)),
                  pl.BlockSpec((K, Nn), lambda i: (0, 0))],
        out_specs=pl.BlockSpec((bm, Nn), lambda i: (i, 0)),
        out_shape=jax.ShapeDtypeStruct((M, Nn), jnp.float32),
    )(a, b)


def _mm_nt_kernel(a_ref, b_ref, o_ref):
    o_ref[...] = lax.dot_general(
        a_ref[...], b_ref[...], (((1,), (1,)), ((), ())),
        preferred_element_type=jnp.float32)


def mm_nt(a, b, bm=512, bn=512):
    M, K = a.shape
    Nn, _ = b.shape
    bm, bn = min(bm, M), min(bn, Nn)
    return pl.pallas_call(
        _mm_nt_kernel,
        grid=(M // bm, Nn // bn),
        in_specs=[pl.BlockSpec((bm, K), lambda i, j: (i, 0)),
                  pl.BlockSpec((bn, K), lambda i, j: (j, 0))],
        out_specs=pl.BlockSpec((bm, bn), lambda i, j: (i, j)),
        out_shape=jax.ShapeDtypeStruct((M, Nn), jnp.float32),
    )(a, b)


def _sc_mesh():
    return plsc.VectorSubcoreMesh(core_axis_name="c", subcore_axis_name="s")


_SC_PARAMS = pltpu.CompilerParams(needs_layout_passes=False)


_EPW = E // 32
_ACC_ROWS = 4224


def _edge_stats_sc(se, de):
    @functools.partial(
        pl.kernel,
        out_type=[jax.ShapeDtypeStruct((32, N), jnp.float32),
                  jax.ShapeDtypeStruct((32, N), jnp.float32)],
        mesh=_sc_mesh(),
        compiler_params=_SC_PARAMS,
        scratch_types=[pltpu.VMEM((_EPW,), jnp.int32),
                       pltpu.VMEM((_EPW,), jnp.int32),
                       pltpu.VMEM((N,), jnp.float32),
                       pltpu.VMEM((N,), jnp.float32),
                       pltpu.SemaphoreType.DMA],
    )
    def body(se_hbm, de_hbm, oself_hbm, oin_hbm, se_v, de_v, accs_v, acci_v,
             sem):
        c = lax.axis_index("c")
        s = lax.axis_index("s")
        wid = s * 2 + c
        base = wid * _EPW

        @pl.loop(0, N, step=16)
        def _(i):
            z = jnp.zeros((16,), jnp.float32)
            accs_v[pl.ds(i, 16)] = z
            acci_v[pl.ds(i, 16)] = z

        pltpu.sync_copy(se_hbm.at[pl.ds(base, _EPW)], se_v)
        pltpu.sync_copy(de_hbm.at[pl.ds(base, _EPW)], de_v)
        ones = jnp.ones((16,), jnp.float32)

        @pl.loop(0, _EPW, step=16)
        def _(j):
            sv = se_v[pl.ds(j, 16)]
            dv = de_v[pl.ds(j, 16)]
            m_self = sv == dv
            plsc.addupdate_scatter(accs_v, [dv], ones, mask=m_self)
            plsc.addupdate_scatter(acci_v, [dv], ones,
                                   mask=jnp.logical_not(m_self))

        pltpu.sync_copy(accs_v, oself_hbm.at[wid])
        pltpu.sync_copy(acci_v, oin_hbm.at[wid])

    return body(se, de)


def _spmv_sc(se, de, u):
    @functools.partial(
        pl.kernel,
        out_type=jax.ShapeDtypeStruct((2 * _ACC_ROWS, 128), jnp.float32),
        mesh=_sc_mesh(),
        compiler_params=_SC_PARAMS,
        scratch_types=[pltpu.VMEM((_EPW,), jnp.int32),
                       pltpu.VMEM((_EPW,), jnp.int32),
                       pltpu.VMEM((16, 128), jnp.int32),
                       pltpu.VMEM((128, 128), jnp.float32),
                       pltpu.VMEM((264, 128), jnp.float32),
                       pltpu.VMEM_SHARED((_ACC_ROWS, 128), jnp.float32),
                       pltpu.SemaphoreType.DMA],
    )
    def body(se_hbm, de_hbm, u_hbm, out_hbm, se_v, de_v, didx_v, rows_v,
             zbuf_v, acc_sh, sem):
        c = lax.axis_index("c")
        s = lax.axis_index("s")
        wid = s * 2 + c
        base = wid * _EPW

        @pl.loop(0, 264, step=1)
        def _(r):
            @pl.loop(0, 128, step=16)
            def _(cc):
                zbuf_v[r, pl.ds(cc, 16)] = jnp.zeros((16,), jnp.float32)

        pltpu.sync_copy(zbuf_v, acc_sh.at[pl.ds(s * 264, 264)])
        plsc.subcore_barrier()

        pltpu.sync_copy(se_hbm.at[pl.ds(base, _EPW)], se_v)
        pltpu.sync_copy(de_hbm.at[pl.ds(base, _EPW)], de_v)

        dummy = jnp.full((16,), 4096, jnp.int32) + s

        @pl.loop(0, _EPW, step=16)
        def _(j):
            sv = se_v[pl.ds(j, 16)]
            dv = de_v[pl.ds(j, 16)]
            dd = jnp.where(sv == dv, dummy, dv)
            didx_v[j // 128, pl.ds(j % 128, 16)] = dd

        @pl.loop(0, 16, step=1)
        def _(k):
            pltpu.async_copy(u_hbm.at[se_v.at[pl.ds(k * 128, 128)]],
                             rows_v, sem).wait()
            pltpu.sync_copy(rows_v, acc_sh.at[didx_v.at[k]], add=True)

        plsc.subcore_barrier()
        pltpu.sync_copy(acc_sh.at[pl.ds(s * 264, 264)],
                        out_hbm.at[pl.ds(c * _ACC_ROWS + s * 264, 264)])

    return body(se, de, u)


def _build_rct_sc(se, de, posk1):
    nw = N // 4

    @functools.partial(
        pl.kernel,
        out_type=[jax.ShapeDtypeStruct((K1 * 8, 128), jnp.int32),
                  jax.ShapeDtypeStruct((K1 * 8, 128), jnp.int32)],
        mesh=_sc_mesh(),
        compiler_params=_SC_PARAMS,
        scratch_types=[pltpu.VMEM((4096,), jnp.int32),
                       pltpu.VMEM((4096,), jnp.int32),
                       pltpu.VMEM((N,), jnp.int32),
                       pltpu.VMEM((4096,), jnp.int32),
                       pltpu.VMEM((4096,), jnp.int32),
                       pltpu.VMEM((128, 128), jnp.int32),
                       pltpu.VMEM((32, 128), jnp.int32),
                       pltpu.VMEM((264, 128), jnp.int32),
                       pltpu.VMEM_SHARED((4224, 128), jnp.int32),
                       pltpu.SemaphoreType.DMA],
    )
    def body(se_hbm, de_hbm, posk_hbm, r0_hbm, ct0_hbm, se_v, de_v, posk_v,
             rks_v, rkd_v, val_v, sidx_v, zbuf_v, chunk_sh, sem):
        c = lax.axis_index("c")
        s = lax.axis_index("s")
        base = s * 4096
        pltpu.sync_copy(posk_hbm, posk_v)
        pltpu.sync_copy(se_hbm.at[pl.ds(base, 4096)], se_v)
        pltpu.sync_copy(de_hbm.at[pl.ds(base, 4096)], de_v)

        one_i = jnp.ones((16,), jnp.int32)
        i16 = lax.iota(jnp.int32, 16)
        neg1 = jnp.full((16,), -1, jnp.int32)
        c127 = jnp.full((16,), 127, jnp.int32)
        c3 = jnp.full((16,), 3, jnp.int32)
        c8 = jnp.full((16,), 8, jnp.int32)

        @pl.loop(0, 264, step=1)
        def _(r):
            @pl.loop(0, 128, step=16)
            def _(cc):
                zbuf_v[r, pl.ds(cc, 16)] = jnp.zeros((16,), jnp.int32)

        @pl.loop(0, 4096, step=16)
        def _(j):
            sv = se_v[pl.ds(j, 16)]
            dv = de_v[pl.ds(j, 16)]
            nonself = sv != dv
            rks = plsc.load_gather(posk_v, [sv])
            rkd = plsc.load_gather(posk_v, [dv])
            rks_v[pl.ds(j, 16)] = jnp.where(nonself, rks, neg1)
            rkd_v[pl.ds(j, 16)] = jnp.where(nonself, rkd, neg1)

        @pl.loop(0, 128, step=1)
        def _(r):
            @pl.loop(0, 128, step=16)
            def _(cc):
                val_v[r, pl.ds(cc, 16)] = jnp.zeros((16,), jnp.int32)

        def one_scan(rk_ref, col_ref, out_hbm, row0):
            pltpu.sync_copy(zbuf_v, chunk_sh.at[pl.ds(s * 264, 264)])
            plsc.subcore_barrier()

            @pl.loop(0, 32, step=1)
            def _(g):
                @pl.loop(0, 128, step=16)
                def _(jj):
                    j = g * 128 + jj
                    rk = rk_ref[pl.ds(j, 16)]
                    col = col_ref[pl.ds(j, 16)]
                    rloc = rk - row0
                    v = jnp.logical_and(rloc >= 0, rloc < 512)
                    flatb = rloc * N + col
                    dummyv = jnp.full((16,), 4096, jnp.int32) + \
                        jnp.bitwise_and(i16 + j, c127)
                    strow = jnp.where(
                        v, lax.shift_right_logical(flatb, 9), dummyv)
                    wlane = jnp.bitwise_and(
                        lax.shift_right_logical(flatb, 2), c127)
                    wlane = jnp.where(v, wlane, i16)
                    bshift = lax.shift_left(
                        one_i, jnp.bitwise_and(flatb, c3) * c8)
                    bval = jnp.where(v, bshift, one_i)
                    plsc.store_scatter(val_v, [i16 + jj, wlane], bval)
                    sidx_v[g, pl.ds(jj, 16)] = strow

                pltpu.sync_copy(val_v, chunk_sh.at[sidx_v.at[g]],
                                add=True)

                @pl.loop(0, 128, step=16)
                def _(jj):
                    j = g * 128 + jj
                    rk = rk_ref[pl.ds(j, 16)]
                    col = col_ref[pl.ds(j, 16)]
                    rloc = rk - row0
                    v = jnp.logical_and(rloc >= 0, rloc < 512)
                    flatb = rloc * N + col
                    wlane = jnp.bitwise_and(
                        lax.shift_right_logical(flatb, 2), c127)
                    wlane = jnp.where(v, wlane, i16)
                    plsc.store_scatter(val_v, [i16 + jj, wlane],
                                       jnp.zeros((16,), jnp.int32))

            plsc.subcore_barrier()
            pltpu.sync_copy(
                chunk_sh.at[pl.ds(s * 256, 256)],
                out_hbm.at[pl.ds(row0 * 8 + s * 256, 256)])
            plsc.subcore_barrier()

        one_scan(rks_v, de_v, r0_hbm, c * 512)
        one_scan(rks_v, de_v, r0_hbm, 1024 + c * 512)
        one_scan(rkd_v, se_v, ct0_hbm, c * 512)
        one_scan(rkd_v, se_v, ct0_hbm, 1024 + c * 512)

    r0w, ct0w = body(se, de, posk1)
    r0 = lax.bitcast_convert_type(r0w, jnp.int8).reshape(K1, N)
    ct0 = lax.bitcast_convert_type(ct0w, jnp.int8).reshape(K1, N)
    return r0, ct0


def _norm0_kernel(sc_ref, id_ref, xw_ref, dis_ref, q_ref, u_ref):
    ones = jnp.ones((32, 1), jnp.float32)
    selfcnt = lax.dot_general(sc_ref[...], ones, (((0,), (0,)), ((), ())),
                              precision=_HI,
                              preferred_element_type=jnp.float32)
    indeg = lax.dot_general(id_ref[...], ones, (((0,), (0,)), ((), ())),
                            precision=_HI,
                            preferred_element_type=jnp.float32)
    newd = jnp.where(selfcnt == 0.0, 2.0, selfcnt)
    deg = indeg + newd
    dis = jnp.where(deg > 0.0, lax.rsqrt(deg), 0.0)
    dis_ref[...] = dis
    q_ref[...] = dis * dis * newd
    u_ref[...] = jnp.concatenate(
        [dis * xw_ref[...], jnp.zeros((N, 128 - CH), jnp.float32)], axis=1)


def norm0(partself, partin, xw):
    col = pl.BlockSpec((N, 1), lambda: (0, 0))
    mat = pl.BlockSpec((N, CH), lambda: (0, 0))
    part = pl.BlockSpec((32, N), lambda: (0, 0))
    return pl.pallas_call(
        _norm0_kernel,
        in_specs=[part, part, mat],
        out_specs=[col, col, pl.BlockSpec((N, 128), lambda: (0, 0))],
        out_shape=[jax.ShapeDtypeStruct((N, 1), jnp.float32),
                   jax.ShapeDtypeStruct((N, 1), jnp.float32),
                   jax.ShapeDtypeStruct((N, 128), jnp.float32)],
    )(partself, partin, xw)


def _cur0_kernel(dis_ref, q_ref, t0_ref, t1_ref, xw_ref, b_ref, p_ref,
                 cur_ref, sr_ref, sc_ref):
    t = t0_ref[...] + t1_ref[...]
    cur = jnp.maximum(
        dis_ref[...] * t + q_ref[...] * xw_ref[...] + b_ref[...],
        0.0)
    cur_ref[...] = cur
    p = p_ref[...]
    pn = p / jnp.sqrt(jnp.sum(p * p))
    sr_ref[...] = jnp.tanh(lax.dot_general(
        pn, cur, (((1,), (1,)), ((), ())), precision=_HI,
        preferred_element_type=jnp.float32))
    sc_ref[...] = jnp.tanh(lax.dot_general(
        cur, pn, (((1,), (1,)), ((), ())), precision=_HI,
        preferred_element_type=jnp.float32))


def cur0_score(dis, q, t0, t1, xw, b, p):
    col = pl.BlockSpec((N, 1), lambda: (0, 0))
    mat = pl.BlockSpec((N, CH), lambda: (0, 0))
    vec = pl.BlockSpec((1, CH), lambda: (0, 0))
    return pl.pallas_call(
        _cur0_kernel,
        in_specs=[col, col, mat, mat, mat, vec, vec],
        out_specs=[mat, pl.BlockSpec((1, N), lambda: (0, 0)), col],
        out_shape=[jax.ShapeDtypeStruct((N, CH), jnp.float32),
                   jax.ShapeDtypeStruct((1, N), jnp.float32),
                   jax.ShapeDtypeStruct((N, 1), jnp.float32)],
    )(dis, q, t0, t1, xw, b.reshape(1, CH), p.reshape(1, CH))


def _topk_kernel(k, bm, sr_ref, sc_ref, srb_ref, scb_ref, pr_ref, pc_ref):
    i = pl.program_id(0)
    n = sr_ref.shape[1]
    s_row = sr_ref[...]
    s_col = sc_ref[...]
    s_row_blk = srb_ref[...].reshape(1, bm)
    idx_blk_l = lax.broadcasted_iota(jnp.int32, (1, bm), 1) + i * bm
    idx_col = lax.broadcasted_iota(jnp.int32, (n, 1), 0)
    gt = (s_col > s_row_blk).astype(jnp.float32)
    tie = jnp.logical_and(s_col == s_row_blk, idx_col < idx_blk_l)
    rank_l = jnp.sum(gt + tie.astype(jnp.float32), axis=0,
                     keepdims=True)
    pr_ref[...] = jnp.where(rank_l < k, rank_l.astype(jnp.int32),
                            -1).reshape(1, 1, bm)
    s_col_blk = scb_ref[...]
    idx_blk_c = lax.broadcasted_iota(jnp.int32, (bm, 1), 0) + i * bm
    idx_row = lax.broadcasted_iota(jnp.int32, (1, n), 1)
    gt2 = (s_row > s_col_blk).astype(jnp.float32)
    tie2 = jnp.logical_and(s_row == s_col_blk, idx_row < idx_blk_c)
    rank_c = jnp.sum(gt2 + tie2.astype(jnp.float32), axis=1,
                     keepdims=True)
    pc_ref[...] = jnp.where(rank_c < k, rank_c.astype(jnp.int32), -1)


def topk_posk(s_row, s_col, k, bm=512):
    n = s_row.shape[1]
    s_row3 = s_row.reshape(n // bm, 1, bm)
    pr3, pc = pl.pallas_call(
        functools.partial(_topk_kernel, k, bm),
        grid=(n // bm,),
        in_specs=[pl.BlockSpec((1, n), lambda i: (0, 0)),
                  pl.BlockSpec((n, 1), lambda i: (0, 0)),
                  pl.BlockSpec((1, 1, bm), lambda i: (i, 0, 0)),
                  pl.BlockSpec((bm, 1), lambda i: (i, 0))],
        out_specs=[pl.BlockSpec((1, 1, bm), lambda i: (i, 0, 0)),
                   pl.BlockSpec((bm, 1), lambda i: (i, 0))],
        out_shape=[jax.ShapeDtypeStruct((n // bm, 1, bm), jnp.int32),
                   jax.ShapeDtypeStruct((n, 1), jnp.int32)],
    )(s_row, s_col, s_row3, s_col)
    return pr3.reshape(1, n), pc


def _gather_kernel(bm, posk_ref, feat_ref, s_ref, o_ref):
    i = pl.program_id(0)
    posk = posk_ref[...]
    rr = lax.broadcasted_iota(jnp.int32, (bm, 1), 0) + i * bm
    oh = (posk == rr).astype(jnp.float32)
    fs = feat_ref[...] * s_ref[...]
    o_ref[...] = jnp.dot(oh, fs, precision=_HI,
                         preferred_element_type=jnp.float32)


def pool_gather(posk_row, feat, s_col, k, bm=512):
    n, f = feat.shape
    return pl.pallas_call(
        functools.partial(_gather_kernel, bm),
        grid=(k // bm,),
        in_specs=[pl.BlockSpec((1, n), lambda i: (0, 0)),
                  pl.BlockSpec((n, f), lambda i: (0, 0)),
                  pl.BlockSpec((n, 1), lambda i: (0, 0))],
        out_specs=pl.BlockSpec((bm, f), lambda i: (i, 0)),
        out_shape=jax.ShapeDtypeStruct((k, f), jnp.float32),
    )(posk_row, feat, s_col)


def _unpool_kernel(bm, posk_ref, cur_ref, res_ref, o_ref):
    posk_blk = posk_ref[...]
    k = cur_ref.shape[0]
    cc = lax.broadcasted_iota(jnp.int32, (1, k), 1)
    oh = (posk_blk == cc).astype(jnp.float32)
    up = jnp.dot(oh, cur_ref[...], precision=_HI,
                 preferred_element_type=jnp.float32)
    o_ref[...] = res_ref[...] + up


def unpool_add(posk_col, cur, res, bm=512):
    n, f = res.shape
    k = cur.shape[0]
    return pl.pallas_call(
        functools.partial(_unpool_kernel, bm),
        grid=(n // bm,),
        in_specs=[pl.BlockSpec((bm, 1), lambda i: (i, 0)),
                  pl.BlockSpec((k, f), lambda i: (0, 0)),
                  pl.BlockSpec((bm, f), lambda i: (i, 0))],
        out_specs=pl.BlockSpec((bm, f), lambda i: (i, 0)),
        out_shape=jax.ShapeDtypeStruct((n, f), jnp.float32),
    )(posk_col, cur, res)


def _addeye_cast_kernel(bm, m_ref, posk_ref, o_ref):
    i = pl.program_id(0)
    posk = posk_ref[...]
    rr = lax.broadcasted_iota(jnp.int32, (bm, 1), 0) + i * bm
    oh = (posk == rr).astype(jnp.float32)
    o_ref[...] = (m_ref[...].astype(jnp.float32) + oh).astype(jnp.bfloat16)


def addeye_cast(m, posk_row, bm=512):
    k, n = m.shape
    return pl.pallas_call(
        functools.partial(_addeye_cast_kernel, bm),
        grid=(k // bm,),
        in_specs=[pl.BlockSpec((bm, n), lambda i: (i, 0)),
                  pl.BlockSpec((1, n), lambda i: (0, 0))],
        out_specs=pl.BlockSpec((bm, n), lambda i: (i, 0)),
        out_shape=jax.ShapeDtypeStruct((k, n), jnp.bfloat16),
    )(m, posk_row)


def _onehot_rows_kernel(bm, posk_ref, o_ref):
    i = pl.program_id(0)
    posk = posk_ref[...]
    rr = lax.broadcasted_iota(jnp.int32, (bm, 1), 0) + i * bm
    o_ref[...] = (posk == rr).astype(jnp.bfloat16)


def onehot_rows(posk_row, k, bm=512):
    n = posk_row.shape[1]
    return pl.pallas_call(
        functools.partial(_onehot_rows_kernel, bm),
        grid=(k // bm,),
        in_specs=[pl.BlockSpec((1, n), lambda i: (0, 0))],
        out_specs=pl.BlockSpec((bm, n), lambda i: (i, 0)),
        out_shape=jax.ShapeDtypeStruct((k, n), jnp.bfloat16),
    )(posk_row)


def _zerodiag_cast_kernel(bm, add_eye, p_ref, o_ref, ob_ref):
    i = pl.program_id(0)
    j = pl.program_id(1)
    bn = p_ref.shape[1]
    rr = lax.broadcasted_iota(jnp.int32, (bm, 1), 0) + i * bm
    cc = lax.broadcasted_iota(jnp.int32, (1, bn), 1) + j * bn
    diag = (rr == cc).astype(jnp.float32)
    v = p_ref[...] * (1.0 - diag)
    o_ref[...] = v
    if add_eye:
        ob_ref[...] = (v + diag).astype(jnp.bfloat16)
    else:
        ob_ref[...] = v.astype(jnp.bfloat16)


def zerodiag(p, add_eye, bm=512):
    k = p.shape[0]
    return pl.pallas_call(
        functools.partial(_zerodiag_cast_kernel, bm, add_eye),
        grid=(k // bm, k // bm),
        in_specs=[pl.BlockSpec((bm, bm), lambda i, j: (i, j))],
        out_specs=[pl.BlockSpec((bm, bm), lambda i, j: (i, j)),
                   pl.BlockSpec((bm, bm), lambda i, j: (i, j))],
        out_shape=[jax.ShapeDtypeStruct((k, k), jnp.float32),
                   jax.ShapeDtypeStruct((k, k), jnp.bfloat16)],
    )(p)


def _degcol_kernel(ap_ref, o_ref):
    k = ap_ref.shape[0]
    ones = jnp.ones((k, 1), jnp.float32)
    o_ref[...] = lax.dot_general(ap_ref[...], ones, (((0,), (0,)), ((), ())),
                                 precision=_HI,
                                 preferred_element_type=jnp.float32)


def degcol(ap, bm=512):
    k = ap.shape[0]
    return pl.pallas_call(
        _degcol_kernel,
        grid=(k // bm,),
        in_specs=[pl.BlockSpec((k, bm), lambda i: (0, i))],
        out_specs=pl.BlockSpec((bm, 1), lambda i: (i, 0)),
        out_shape=jax.ShapeDtypeStruct((k, 1), jnp.float32),
    )(ap)


def _gcn_kernel(bm, relu, score, ap_ref, cs_ref, v_ref, csb_ref, vb_ref,
                b_ref, p_ref, o_ref, sr_ref, sc_ref):
    deg = cs_ref[...] + 2.0
    dis = jnp.where(deg > 0.0, lax.rsqrt(deg), 0.0)
    w = dis * v_ref[...]
    t_blk = lax.dot_general(ap_ref[...], w, (((0,), (0,)), ((), ())),
                            precision=_HI,
                            preferred_element_type=jnp.float32)
    degb = csb_ref[...] + 2.0
    dis_blk = jnp.where(degb > 0.0, lax.rsqrt(degb), 0.0)
    w_blk = dis_blk * vb_ref[...]
    o = dis_blk * (t_blk + 2.0 * w_blk) + b_ref[...]
    if relu:
        o = jnp.maximum(o, 0.0)
    o_ref[...] = o
    if score:
        p = p_ref[...]
        pn = p / jnp.sqrt(jnp.sum(p * p))
        sr_ref[...] = jnp.tanh(lax.dot_general(
            pn, o, (((1,), (1,)), ((), ())), precision=_HI,
            preferred_element_type=jnp.float32)).reshape(1, 1, bm)
        sc_ref[...] = jnp.tanh(lax.dot_general(
            o, pn, (((1,), (1,)), ((), ())), precision=_HI,
            preferred_element_type=jnp.float32))


def gcn_dense(ap, cs_col, v, b, p=None, relu=True, bm=512):
    k, f = v.shape
    score = p is not None
    if p is None:
        p = jnp.zeros((CH,), jnp.float32)
    outs = pl.pallas_call(
        functools.partial(_gcn_kernel, bm, relu, score),
        grid=(k // bm,),
        in_specs=[pl.BlockSpec((k, bm), lambda i: (0, i)),
                  pl.BlockSpec((k, 1), lambda i: (0, 0)),
                  pl.BlockSpec((k, f), lambda i: (0, 0)),
                  pl.BlockSpec((bm, 1), lambda i: (i, 0)),
                  pl.BlockSpec((bm, f), lambda i: (i, 0)),
                  pl.BlockSpec((1, f), lambda i: (0, 0)),
                  pl.BlockSpec((1, CH), lambda i: (0, 0))],
        out_specs=[pl.BlockSpec((bm, f), lambda i: (i, 0)),
                   pl.BlockSpec((1, 1, bm), lambda i: (i, 0, 0)),
                   pl.BlockSpec((bm, 1), lambda i: (i, 0))],
        out_shape=[jax.ShapeDtypeStruct((k, f), jnp.float32),
                   jax.ShapeDtypeStruct((k // bm, 1, bm), jnp.float32),
                   jax.ShapeDtypeStruct((k, 1), jnp.float32)],
    )(ap, cs_col, v, cs_col, v, b.reshape(1, f), p.reshape(1, CH))
    if score:
        return outs[0], outs[1].reshape(1, k), outs[2]
    return outs[0]


def _scale_kernel(dis_ref, c_ref, u_ref):
    n, f = c_ref.shape
    u_ref[...] = jnp.concatenate(
        [dis_ref[...] * c_ref[...], jnp.zeros((n, 128 - f), jnp.float32)],
        axis=1)


def scale_rows(dis_col, c):
    n, f = c.shape
    return pl.pallas_call(
        _scale_kernel,
        in_specs=[pl.BlockSpec((n, 1), lambda: (0, 0)),
                  pl.BlockSpec((n, f), lambda: (0, 0))],
        out_specs=pl.BlockSpec((n, 128), lambda: (0, 0)),
        out_shape=jax.ShapeDtypeStruct((n, 128), jnp.float32),
    )(dis_col, c)


def _scale_add_kernel(dis_ref, q_ref, t0_ref, t1_ref, c_ref, z_ref):
    t = t0_ref[...] + t1_ref[...]
    z_ref[...] = dis_ref[...] * t + q_ref[...] * c_ref[...]


def scale_add(dis_col, q_col, t0, t1, c):
    n, f = c.shape
    col = pl.BlockSpec((n, 1), lambda: (0, 0))
    mat = pl.BlockSpec((n, f), lambda: (0, 0))
    return pl.pallas_call(
        _scale_add_kernel,
        in_specs=[col, col, mat, mat, mat],
        out_specs=mat,
        out_shape=jax.ShapeDtypeStruct((n, f), jnp.float32),
    )(dis_col, q_col, t0, t1, c)


def _final_kernel(z_ref, wu_ref, bu_ref, g_ref, be_ref, batch_ref, x_ref,
                  l0_ref, l1_ref, lb_ref, o_ref):
    h = jnp.dot(z_ref[...], wu_ref[...], precision=_HI,
                preferred_element_type=jnp.float32) + bu_ref[...]
    h = h * g_ref[...] + be_ref[...]
    h = jnp.maximum(h, 0.0)
    batch = batch_ref[...]
    bb = lax.broadcasted_iota(jnp.int32, (NB, 1), 0)
    S = (batch == bb).astype(jnp.float32)
    pooled0 = jnp.dot(S, x_ref[...], precision=_HI,
                      preferred_element_type=jnp.float32)
    pooled1 = jnp.dot(S, h, precision=_HI,
                      preferred_element_type=jnp.float32)
    o_ref[...] = (jnp.dot(pooled0, l0_ref[...], precision=_HI,
                          preferred_element_type=jnp.float32)
                  + jnp.dot(pooled1, l1_ref[...], precision=_HI,
                            preferred_element_type=jnp.float32)
                  + lb_ref[...])


def final_stage(z, Wu1, bu1, gscaled, bn_beta, batch, x, L0W, L1W, lb):
    nh = Wu1.shape[1]
    fs = lambda shp: pl.BlockSpec(shp, lambda: (0, 0))
    return pl.pallas_call(
        _final_kernel,
        in_specs=[fs((N, CH)), fs((CH, nh)), fs((1, nh)), fs((1, nh)),
                  fs((1, nh)), fs((1, N)), fs((N, D_IN)), fs((D_IN, OUT)),
                  fs((nh, OUT)), fs((1, OUT))],
        out_specs=fs((NB, OUT)),
        out_shape=jax.ShapeDtypeStruct((NB, OUT), jnp.float32),
    )(z, Wu1, bu1.reshape(1, nh), gscaled.reshape(1, nh),
      bn_beta.reshape(1, nh), batch.reshape(1, N), x, L0W, L1W,
      lb.reshape(1, OUT))


def kernel(x, edge_index, batch, W0, b0, W1, b1, W2, b2, p0, p1, Wu0, bu0,
           Wu1, bu1, bn_gamma, bn_beta, L0W, L0b, L1W, L1b):
    se, de = edge_index[0], edge_index[1]

    xw0 = mm_nn(x, W0)
    partself, partin = _edge_stats_sc(se, de)
    dis0, q0, u1 = norm0(partself, partin, xw0)
    tacc = _spmv_sc(se, de, u1)
    t1a = tacc[0:N, 0:CH]
    t1b = tacc[_ACC_ROWS:_ACC_ROWS + N, 0:CH]
    cur0, s1r, s1c = cur0_score(dis0, q0, t1a, t1b, xw0, b0, p0)

    posk1r, posk1c = topk_posk(s1r, s1c, K1)
    x1 = pool_gather(posk1r, cur0, s1c, K1)

    r0, ct0 = _build_rct_sc(se, de, posk1r.reshape(N))
    rb = addeye_cast(r0, posk1r)
    ctb = addeye_cast(ct0, posk1r)
    p_mat = mm_nt(rb, ctb)
    ap1, m2b = zerodiag(p_mat, add_eye=True)
    cs1 = degcol(ap1)

    v1 = mm_nn(x1, W1)
    cur1, s2r, s2c = gcn_dense(ap1, cs1, v1, b1, p=p1, relu=True)

    posk2r, posk2c = topk_posk(s2r, s2c, K2)
    x2 = pool_gather(posk2r, cur1, s2c, K2)

    o2 = onehot_rows(posk2r, K2)
    g2 = mm_nn(o2, m2b)
    h2 = mm_nt(m2b, o2)
    p2 = mm_nn(g2.astype(jnp.bfloat16), h2.astype(jnp.bfloat16))
    ap2, _ = zerodiag(p2, add_eye=False)
    cs2 = degcol(ap2)

    v2 = mm_nn(x2, W2)
    cur2 = gcn_dense(ap2, cs2, v2, b2, relu=True)

    mid = unpool_add(posk2c, cur2, cur1)
    vu0 = mm_nn(mid, Wu0)
    curu1 = gcn_dense(ap1, cs1, vu0, bu0, relu=True)

    full = unpool_add(posk1c, curu1, cur0)
    u2 = scale_rows(dis0, full)
    tacc2 = _spmv_sc(se, de, u2)
    z = scale_add(dis0, q0, tacc2[0:N, 0:CH],
                  tacc2[_ACC_ROWS:_ACC_ROWS + N, 0:CH], full)

    gscaled = bn_gamma / jnp.sqrt(1.0 + 1e-05)
    lb = L0b + L1b
    return final_stage(z, Wu1, bu1, gscaled, bn_beta, batch, x, L0W, L1W, lb)


# speedup vs baseline: 3.9873x; 3.9873x over previous
import functools
import jax
import jax.numpy as jnp
from jax import lax
from jax.experimental import pallas as pl
from jax.experimental.pallas import tpu as pltpu
from jax.experimental.pallas import tpu_sc as plsc

N = 4096
E = 65536
D_IN = 128
CH = 32
OUT = 16
NB = 64
K1 = 2048
K2 = 1024

_HI = lax.Precision.HIGHEST


def _mm_nn_kernel(prec, a_ref, b_ref, o_ref):
    o_ref[...] = jnp.dot(a_ref[...], b_ref[...], precision=prec,
                         preferred_element_type=jnp.float32)


def mm_nn(a, b, bm=512):
    prec = _HI if a.dtype == jnp.float32 else None
    M, K = a.shape
    _, Nn = b.shape
    bm = min(bm, M)
    return pl.pallas_call(
        functools.partial(_mm_nn_kernel, prec),
        grid=(M // bm,),
        in_specs=[pl.BlockSpec((bm, K), lambda i: (i, 0)),
                  pl.BlockSpec((K, Nn), lambda i: (0, 0))],
        out_specs=pl.BlockSpec((bm, Nn), lambda i: (i, 0)),
        out_shape=jax.ShapeDtypeStruct((M, Nn), jnp.float32),
    )(a, b)


def _mm_nt_kernel(a_ref, b_ref, o_ref):
    o_ref[...] = lax.dot_general(
        a_ref[...], b_ref[...], (((1,), (1,)), ((), ())),
        preferred_element_type=jnp.float32)


def mm_nt(a, b, bm=512, bn=512):
    M, K = a.shape
    Nn, _ = b.shape
    bm, bn = min(bm, M), min(bn, Nn)
    return pl.pallas_call(
        _mm_nt_kernel,
        grid=(M // bm, Nn // bn),
        in_specs=[pl.BlockSpec((bm, K), lambda i, j: (i, 0)),
                  pl.BlockSpec((bn, K), lambda i, j: (j, 0))],
        out_specs=pl.BlockSpec((bm, bn), lambda i, j: (i, j)),
        out_shape=jax.ShapeDtypeStruct((M, Nn), jnp.float32),
    )(a, b)


def _sc_mesh():
    return plsc.VectorSubcoreMesh(core_axis_name="c", subcore_axis_name="s")


_SC_PARAMS = pltpu.CompilerParams(needs_layout_passes=False)


_EPW = E // 32
_ACC_ROWS = 4224


def _edge_stats_sc(se, de):
    @functools.partial(
        pl.kernel,
        out_type=[jax.ShapeDtypeStruct((32, N), jnp.float32),
                  jax.ShapeDtypeStruct((32, N), jnp.float32)],
        mesh=_sc_mesh(),
        compiler_params=_SC_PARAMS,
        scratch_types=[pltpu.VMEM((_EPW,), jnp.int32),
                       pltpu.VMEM((_EPW,), jnp.int32),
                       pltpu.VMEM((N,), jnp.float32),
                       pltpu.VMEM((N,), jnp.float32),
                       pltpu.SemaphoreType.DMA],
    )
    def body(se_hbm, de_hbm, oself_hbm, oin_hbm, se_v, de_v, accs_v, acci_v,
             sem):
        c = lax.axis_index("c")
        s = lax.axis_index("s")
        wid = s * 2 + c
        base = wid * _EPW

        @pl.loop(0, N, step=16)
        def _(i):
            z = jnp.zeros((16,), jnp.float32)
            accs_v[pl.ds(i, 16)] = z
            acci_v[pl.ds(i, 16)] = z

        pltpu.sync_copy(se_hbm.at[pl.ds(base, _EPW)], se_v)
        pltpu.sync_copy(de_hbm.at[pl.ds(base, _EPW)], de_v)
        ones = jnp.ones((16,), jnp.float32)

        @pl.loop(0, _EPW, step=16)
        def _(j):
            sv = se_v[pl.ds(j, 16)]
            dv = de_v[pl.ds(j, 16)]
            m_self = sv == dv
            plsc.addupdate_scatter(accs_v, [dv], ones, mask=m_self)
            plsc.addupdate_scatter(acci_v, [dv], ones,
                                   mask=jnp.logical_not(m_self))

        pltpu.sync_copy(accs_v, oself_hbm.at[wid])
        pltpu.sync_copy(acci_v, oin_hbm.at[wid])

    return body(se, de)


def _spmv_sc(se, de, u):
    @functools.partial(
        pl.kernel,
        out_type=jax.ShapeDtypeStruct((2 * _ACC_ROWS, 128), jnp.float32),
        mesh=_sc_mesh(),
        compiler_params=_SC_PARAMS,
        scratch_types=[pltpu.VMEM((_EPW,), jnp.int32),
                       pltpu.VMEM((_EPW,), jnp.int32),
                       pltpu.VMEM((16, 128), jnp.int32),
                       pltpu.VMEM((128, 128), jnp.float32),
                       pltpu.VMEM((264, 128), jnp.float32),
                       pltpu.VMEM_SHARED((_ACC_ROWS, 128), jnp.float32),
                       pltpu.SemaphoreType.DMA],
    )
    def body(se_hbm, de_hbm, u_hbm, out_hbm, se_v, de_v, didx_v, rows_v,
             zbuf_v, acc_sh, sem):
        c = lax.axis_index("c")
        s = lax.axis_index("s")
        wid = s * 2 + c
        base = wid * _EPW

        @pl.loop(0, 264, step=1)
        def _(r):
            @pl.loop(0, 128, step=16)
            def _(cc):
                zbuf_v[r, pl.ds(cc, 16)] = jnp.zeros((16,), jnp.float32)

        pltpu.sync_copy(zbuf_v, acc_sh.at[pl.ds(s * 264, 264)])
        plsc.subcore_barrier()

        pltpu.sync_copy(se_hbm.at[pl.ds(base, _EPW)], se_v)
        pltpu.sync_copy(de_hbm.at[pl.ds(base, _EPW)], de_v)

        dummy = jnp.full((16,), 4096, jnp.int32) + s

        @pl.loop(0, _EPW, step=16)
        def _(j):
            sv = se_v[pl.ds(j, 16)]
            dv = de_v[pl.ds(j, 16)]
            dd = jnp.where(sv == dv, dummy, dv)
            didx_v[j // 128, pl.ds(j % 128, 16)] = dd

        @pl.loop(0, 16, step=1)
        def _(k):
            pltpu.async_copy(u_hbm.at[se_v.at[pl.ds(k * 128, 128)]],
                             rows_v, sem).wait()
            pltpu.sync_copy(rows_v, acc_sh.at[didx_v.at[k]], add=True)

        plsc.subcore_barrier()
        pltpu.sync_copy(acc_sh.at[pl.ds(s * 264, 264)],
                        out_hbm.at[pl.ds(c * _ACC_ROWS + s * 264, 264)])

    return body(se, de, u)


def _build_rct_sc(se, de, posk1):
    nw = N // 4

    @functools.partial(
        pl.kernel,
        out_type=[jax.ShapeDtypeStruct((K1, nw), jnp.int32),
                  jax.ShapeDtypeStruct((K1, nw), jnp.int32)],
        mesh=_sc_mesh(),
        compiler_params=_SC_PARAMS,
        scratch_types=[pltpu.VMEM((2048,), jnp.int32),
                       pltpu.VMEM((2048,), jnp.int32),
                       pltpu.VMEM((N,), jnp.int32),
                       pltpu.VMEM((64, nw), jnp.int32),
                       pltpu.SemaphoreType.DMA],
    )
    def body(se_hbm, de_hbm, posk_hbm, r0_hbm, ct0_hbm, se_v, de_v, posk_v,
             chunk_v, sem):
        c = lax.axis_index("c")
        s = lax.axis_index("s")
        wid = s * 2 + c
        row0 = wid * 64
        pltpu.sync_copy(posk_hbm, posk_v)

        one_i = jnp.ones((16,), jnp.int32)
        i16 = lax.iota(jnp.int32, 16)
        zero16 = jnp.zeros((16,), jnp.int32)
        c127 = jnp.full((16,), 1023, jnp.int32)
        c3 = jnp.full((16,), 3, jnp.int32)
        c8 = jnp.full((16,), 8, jnp.int32)

        def one_matrix(out_hbm, use_src_for_row):
            @pl.loop(0, 64, step=1)
            def _(r):
                @pl.loop(0, nw, step=16)
                def _(cc):
                    chunk_v[r, pl.ds(cc, 16)] = zero16

            @pl.loop(0, E, step=2048)
            def _(eb):
                pltpu.sync_copy(se_hbm.at[pl.ds(eb, 2048)], se_v)
                pltpu.sync_copy(de_hbm.at[pl.ds(eb, 2048)], de_v)

                @pl.loop(0, 2048, step=16)
                def _(j):
                    sv = se_v[pl.ds(j, 16)]
                    dv = de_v[pl.ds(j, 16)]
                    if use_src_for_row:
                        rk = plsc.load_gather(posk_v, [sv])
                        col = dv
                    else:
                        rk = plsc.load_gather(posk_v, [dv])
                        col = sv
                    rloc = rk - row0
                    valid = jnp.logical_and(
                        jnp.logical_and(rloc >= 0, rloc < 64),
                        sv != dv)
                    rsafe = jnp.where(valid, rloc, zero16)
                    word = jnp.bitwise_and(
                        lax.shift_right_logical(col, 2), c127)
                    bval = lax.shift_left(
                        one_i, jnp.bitwise_and(col, c3) * c8)
                    plsc.addupdate_scatter(chunk_v, [rsafe, word], bval,
                                           mask=valid)

            pltpu.sync_copy(chunk_v, out_hbm.at[pl.ds(row0, 64)])

        one_matrix(r0_hbm, True)
        one_matrix(ct0_hbm, False)

    r0w, ct0w = body(se, de, posk1)
    r0 = lax.bitcast_convert_type(r0w, jnp.int8).reshape(K1, N)
    ct0 = lax.bitcast_convert_type(ct0w, jnp.int8).reshape(K1, N)
    return r0, ct0


def _norm0_kernel(sc_ref, id_ref, xw_ref, dis_ref, q_ref, u_ref):
    ones = jnp.ones((32, 1), jnp.float32)
    selfcnt = lax.dot_general(sc_ref[...], ones, (((0,), (0,)), ((), ())),
                              precision=_HI,
                              preferred_element_type=jnp.float32)
    indeg = lax.dot_general(id_ref[...], ones, (((0,), (0,)), ((), ())),
                            precision=_HI,
                            preferred_element_type=jnp.float32)
    newd = jnp.where(selfcnt == 0.0, 2.0, selfcnt)
    deg = indeg + newd
    dis = jnp.where(deg > 0.0, lax.rsqrt(deg), 0.0)
    dis_ref[...] = dis
    q_ref[...] = dis * dis * newd
    u_ref[...] = jnp.concatenate(
        [dis * xw_ref[...], jnp.zeros((N, 128 - CH), jnp.float32)], axis=1)


def norm0(partself, partin, xw):
    col = pl.BlockSpec((N, 1), lambda: (0, 0))
    mat = pl.BlockSpec((N, CH), lambda: (0, 0))
    part = pl.BlockSpec((32, N), lambda: (0, 0))
    return pl.pallas_call(
        _norm0_kernel,
        in_specs=[part, part, mat],
        out_specs=[col, col, pl.BlockSpec((N, 128), lambda: (0, 0))],
        out_shape=[jax.ShapeDtypeStruct((N, 1), jnp.float32),
                   jax.ShapeDtypeStruct((N, 1), jnp.float32),
                   jax.ShapeDtypeStruct((N, 128), jnp.float32)],
    )(partself, partin, xw)


def _cur0_kernel(dis_ref, q_ref, t0_ref, t1_ref, xw_ref, b_ref, p_ref,
                 cur_ref, sr_ref, sc_ref):
    t = t0_ref[...] + t1_ref[...]
    cur = jnp.maximum(
        dis_ref[...] * t + q_ref[...] * xw_ref[...] + b_ref[...],
        0.0)
    cur_ref[...] = cur
    p = p_ref[...]
    pn = p / jnp.sqrt(jnp.sum(p * p))
    sr_ref[...] = jnp.tanh(lax.dot_general(
        pn, cur, (((1,), (1,)), ((), ())), precision=_HI,
        preferred_element_type=jnp.float32))
    sc_ref[...] = jnp.tanh(lax.dot_general(
        cur, pn, (((1,), (1,)), ((), ())), precision=_HI,
        preferred_element_type=jnp.float32))


def cur0_score(dis, q, t0, t1, xw, b, p):
    col = pl.BlockSpec((N, 1), lambda: (0, 0))
    mat = pl.BlockSpec((N, CH), lambda: (0, 0))
    vec = pl.BlockSpec((1, CH), lambda: (0, 0))
    return pl.pallas_call(
        _cur0_kernel,
        in_specs=[col, col, mat, mat, mat, vec, vec],
        out_specs=[mat, pl.BlockSpec((1, N), lambda: (0, 0)), col],
        out_shape=[jax.ShapeDtypeStruct((N, CH), jnp.float32),
                   jax.ShapeDtypeStruct((1, N), jnp.float32),
                   jax.ShapeDtypeStruct((N, 1), jnp.float32)],
    )(dis, q, t0, t1, xw, b.reshape(1, CH), p.reshape(1, CH))


def _topk_kernel(k, bm, sr_ref, sc_ref, srb_ref, scb_ref, pr_ref, pc_ref):
    i = pl.program_id(0)
    n = sr_ref.shape[1]
    s_row = sr_ref[...]
    s_col = sc_ref[...]
    s_row_blk = srb_ref[...].reshape(1, bm)
    idx_blk_l = lax.broadcasted_iota(jnp.int32, (1, bm), 1) + i * bm
    idx_col = lax.broadcasted_iota(jnp.int32, (n, 1), 0)
    gt = (s_col > s_row_blk).astype(jnp.float32)
    tie = jnp.logical_and(s_col == s_row_blk, idx_col < idx_blk_l)
    rank_l = jnp.sum(gt + tie.astype(jnp.float32), axis=0,
                     keepdims=True)
    pr_ref[...] = jnp.where(rank_l < k, rank_l.astype(jnp.int32),
                            -1).reshape(1, 1, bm)
    s_col_blk = scb_ref[...]
    idx_blk_c = lax.broadcasted_iota(jnp.int32, (bm, 1), 0) + i * bm
    idx_row = lax.broadcasted_iota(jnp.int32, (1, n), 1)
    gt2 = (s_row > s_col_blk).astype(jnp.float32)
    tie2 = jnp.logical_and(s_row == s_col_blk, idx_row < idx_blk_c)
    rank_c = jnp.sum(gt2 + tie2.astype(jnp.float32), axis=1,
                     keepdims=True)
    pc_ref[...] = jnp.where(rank_c < k, rank_c.astype(jnp.int32), -1)


def topk_posk(s_row, s_col, k, bm=512):
    n = s_row.shape[1]
    s_row3 = s_row.reshape(n // bm, 1, bm)
    pr3, pc = pl.pallas_call(
        functools.partial(_topk_kernel, k, bm),
        grid=(n // bm,),
        in_specs=[pl.BlockSpec((1, n), lambda i: (0, 0)),
                  pl.BlockSpec((n, 1), lambda i: (0, 0)),
                  pl.BlockSpec((1, 1, bm), lambda i: (i, 0, 0)),
                  pl.BlockSpec((bm, 1), lambda i: (i, 0))],
        out_specs=[pl.BlockSpec((1, 1, bm), lambda i: (i, 0, 0)),
                   pl.BlockSpec((bm, 1), lambda i: (i, 0))],
        out_shape=[jax.ShapeDtypeStruct((n // bm, 1, bm), jnp.int32),
                   jax.ShapeDtypeStruct((n, 1), jnp.int32)],
    )(s_row, s_col, s_row3, s_col)
    return pr3.reshape(1, n), pc


def _gather_kernel(bm, posk_ref, feat_ref, s_ref, o_ref):
    i = pl.program_id(0)
    posk = posk_ref[...]
    rr = lax.broadcasted_iota(jnp.int32, (bm, 1), 0) + i * bm
    oh = (posk == rr).astype(jnp.float32)
    fs = feat_ref[...] * s_ref[...]
    o_ref[...] = jnp.dot(oh, fs, precision=_HI,
                         preferred_element_type=jnp.float32)


def pool_gather(posk_row, feat, s_col, k, bm=512):
    n, f = feat.shape
    return pl.pallas_call(
        functools.partial(_gather_kernel, bm),
        grid=(k // bm,),
        in_specs=[pl.BlockSpec((1, n), lambda i: (0, 0)),
                  pl.BlockSpec((n, f), lambda i: (0, 0)),
                  pl.BlockSpec((n, 1), lambda i: (0, 0))],
        out_specs=pl.BlockSpec((bm, f), lambda i: (i, 0)),
        out_shape=jax.ShapeDtypeStruct((k, f), jnp.float32),
    )(posk_row, feat, s_col)


def _unpool_kernel(bm, posk_ref, cur_ref, res_ref, o_ref):
    posk_blk = posk_ref[...]
    k = cur_ref.shape[0]
    cc = lax.broadcasted_iota(jnp.int32, (1, k), 1)
    oh = (posk_blk == cc).astype(jnp.float32)
    up = jnp.dot(oh, cur_ref[...], precision=_HI,
                 preferred_element_type=jnp.float32)
    o_ref[...] = res_ref[...] + up


def unpool_add(posk_col, cur, res, bm=512):
    n, f = res.shape
    k = cur.shape[0]
    return pl.pallas_call(
        functools.partial(_unpool_kernel, bm),
        grid=(n // bm,),
        in_specs=[pl.BlockSpec((bm, 1), lambda i: (i, 0)),
                  pl.BlockSpec((k, f), lambda i: (0, 0)),
                  pl.BlockSpec((bm, f), lambda i: (i, 0))],
        out_specs=pl.BlockSpec((bm, f), lambda i: (i, 0)),
        out_shape=jax.ShapeDtypeStruct((n, f), jnp.float32),
    )(posk_col, cur, res)


def _addeye_cast_kernel(bm, m_ref, posk_ref, o_ref):
    i = pl.program_id(0)
    posk = posk_ref[...]
    rr = lax.broadcasted_iota(jnp.int32, (bm, 1), 0) + i * bm
    oh = (posk == rr).astype(jnp.float32)
    o_ref[...] = (m_ref[...].astype(jnp.float32) + oh).astype(jnp.bfloat16)


def addeye_cast(m, posk_row, bm=512):
    k, n = m.shape
    return pl.pallas_call(
        functools.partial(_addeye_cast_kernel, bm),
        grid=(k // bm,),
        in_specs=[pl.BlockSpec((bm, n), lambda i: (i, 0)),
                  pl.BlockSpec((1, n), lambda i: (0, 0))],
        out_specs=pl.BlockSpec((bm, n), lambda i: (i, 0)),
        out_shape=jax.ShapeDtypeStruct((k, n), jnp.bfloat16),
    )(m, posk_row)


def _onehot_rows_kernel(bm, posk_ref, o_ref):
    i = pl.program_id(0)
    posk = posk_ref[...]
    rr = lax.broadcasted_iota(jnp.int32, (bm, 1), 0) + i * bm
    o_ref[...] = (posk == rr).astype(jnp.bfloat16)


def onehot_rows(posk_row, k, bm=512):
    n = posk_row.shape[1]
    return pl.pallas_call(
        functools.partial(_onehot_rows_kernel, bm),
        grid=(k // bm,),
        in_specs=[pl.BlockSpec((1, n), lambda i: (0, 0))],
        out_specs=pl.BlockSpec((bm, n), lambda i: (i, 0)),
        out_shape=jax.ShapeDtypeStruct((k, n), jnp.bfloat16),
    )(posk_row)


def _zerodiag_cast_kernel(bm, add_eye, p_ref, o_ref, ob_ref):
    i = pl.program_id(0)
    j = pl.program_id(1)
    bn = p_ref.shape[1]
    rr = lax.broadcasted_iota(jnp.int32, (bm, 1), 0) + i * bm
    cc = lax.broadcasted_iota(jnp.int32, (1, bn), 1) + j * bn
    diag = (rr == cc).astype(jnp.float32)
    v = p_ref[...] * (1.0 - diag)
    o_ref[...] = v
    if add_eye:
        ob_ref[...] = (v + diag).astype(jnp.bfloat16)
    else:
        ob_ref[...] = v.astype(jnp.bfloat16)


def zerodiag(p, add_eye, bm=512):
    k = p.shape[0]
    return pl.pallas_call(
        functools.partial(_zerodiag_cast_kernel, bm, add_eye),
        grid=(k // bm, k // bm),
        in_specs=[pl.BlockSpec((bm, bm), lambda i, j: (i, j))],
        out_specs=[pl.BlockSpec((bm, bm), lambda i, j: (i, j)),
                   pl.BlockSpec((bm, bm), lambda i, j: (i, j))],
        out_shape=[jax.ShapeDtypeStruct((k, k), jnp.float32),
                   jax.ShapeDtypeStruct((k, k), jnp.bfloat16)],
    )(p)


def _degcol_kernel(ap_ref, o_ref):
    k = ap_ref.shape[0]
    ones = jnp.ones((k, 1), jnp.float32)
    o_ref[...] = lax.dot_general(ap_ref[...], ones, (((0,), (0,)), ((), ())),
                                 precision=_HI,
                                 preferred_element_type=jnp.float32)


def degcol(ap, bm=512):
    k = ap.shape[0]
    return pl.pallas_call(
        _degcol_kernel,
        grid=(k // bm,),
        in_specs=[pl.BlockSpec((k, bm), lambda i: (0, i))],
        out_specs=pl.BlockSpec((bm, 1), lambda i: (i, 0)),
        out_shape=jax.ShapeDtypeStruct((k, 1), jnp.float32),
    )(ap)


def _gcn_kernel(bm, relu, score, ap_ref, cs_ref, v_ref, csb_ref, vb_ref,
                b_ref, p_ref, o_ref, sr_ref, sc_ref):
    deg = cs_ref[...] + 2.0
    dis = jnp.where(deg > 0.0, lax.rsqrt(deg), 0.0)
    w = dis * v_ref[...]
    t_blk = lax.dot_general(ap_ref[...], w, (((0,), (0,)), ((), ())),
                            precision=_HI,
                            preferred_element_type=jnp.float32)
    degb = csb_ref[...] + 2.0
    dis_blk = jnp.where(degb > 0.0, lax.rsqrt(degb), 0.0)
    w_blk = dis_blk * vb_ref[...]
    o = dis_blk * (t_blk + 2.0 * w_blk) + b_ref[...]
    if relu:
        o = jnp.maximum(o, 0.0)
    o_ref[...] = o
    if score:
        p = p_ref[...]
        pn = p / jnp.sqrt(jnp.sum(p * p))
        sr_ref[...] = jnp.tanh(lax.dot_general(
            pn, o, (((1,), (1,)), ((), ())), precision=_HI,
            preferred_element_type=jnp.float32)).reshape(1, 1, bm)
        sc_ref[...] = jnp.tanh(lax.dot_general(
            o, pn, (((1,), (1,)), ((), ())), precision=_HI,
            preferred_element_type=jnp.float32))


def gcn_dense(ap, cs_col, v, b, p=None, relu=True, bm=512):
    k, f = v.shape
    score = p is not None
    if p is None:
        p = jnp.zeros((CH,), jnp.float32)
    outs = pl.pallas_call(
        functools.partial(_gcn_kernel, bm, relu, score),
        grid=(k // bm,),
        in_specs=[pl.BlockSpec((k, bm), lambda i: (0, i)),
                  pl.BlockSpec((k, 1), lambda i: (0, 0)),
                  pl.BlockSpec((k, f), lambda i: (0, 0)),
                  pl.BlockSpec((bm, 1), lambda i: (i, 0)),
                  pl.BlockSpec((bm, f), lambda i: (i, 0)),
                  pl.BlockSpec((1, f), lambda i: (0, 0)),
                  pl.BlockSpec((1, CH), lambda i: (0, 0))],
        out_specs=[pl.BlockSpec((bm, f), lambda i: (i, 0)),
                   pl.BlockSpec((1, 1, bm), lambda i: (i, 0, 0)),
                   pl.BlockSpec((bm, 1), lambda i: (i, 0))],
        out_shape=[jax.ShapeDtypeStruct((k, f), jnp.float32),
                   jax.ShapeDtypeStruct((k // bm, 1, bm), jnp.float32),
                   jax.ShapeDtypeStruct((k, 1), jnp.float32)],
    )(ap, cs_col, v, cs_col, v, b.reshape(1, f), p.reshape(1, CH))
    if score:
        return outs[0], outs[1].reshape(1, k), outs[2]
    return outs[0]


def _scale_kernel(dis_ref, c_ref, u_ref):
    n, f = c_ref.shape
    u_ref[...] = jnp.concatenate(
        [dis_ref[...] * c_ref[...], jnp.zeros((n, 128 - f), jnp.float32)],
        axis=1)


def scale_rows(dis_col, c):
    n, f = c.shape
    return pl.pallas_call(
        _scale_kernel,
        in_specs=[pl.BlockSpec((n, 1), lambda: (0, 0)),
                  pl.BlockSpec((n, f), lambda: (0, 0))],
        out_specs=pl.BlockSpec((n, 128), lambda: (0, 0)),
        out_shape=jax.ShapeDtypeStruct((n, 128), jnp.float32),
    )(dis_col, c)


def _scale_add_kernel(dis_ref, q_ref, t0_ref, t1_ref, c_ref, z_ref):
    t = t0_ref[...] + t1_ref[...]
    z_ref[...] = dis_ref[...] * t + q_ref[...] * c_ref[...]


def scale_add(dis_col, q_col, t0, t1, c):
    n, f = c.shape
    col = pl.BlockSpec((n, 1), lambda: (0, 0))
    mat = pl.BlockSpec((n, f), lambda: (0, 0))
    return pl.pallas_call(
        _scale_add_kernel,
        in_specs=[col, col, mat, mat, mat],
        out_specs=mat,
        out_shape=jax.ShapeDtypeStruct((n, f), jnp.float32),
    )(dis_col, q_col, t0, t1, c)


def _final_kernel(z_ref, wu_ref, bu_ref, g_ref, be_ref, batch_ref, x_ref,
                  l0_ref, l1_ref, lb_ref, o_ref):
    h = jnp.dot(z_ref[...], wu_ref[...], precision=_HI,
                preferred_element_type=jnp.float32) + bu_ref[...]
    h = h * g_ref[...] + be_ref[...]
    h = jnp.maximum(h, 0.0)
    batch = batch_ref[...]
    bb = lax.broadcasted_iota(jnp.int32, (NB, 1), 0)
    S = (batch == bb).astype(jnp.float32)
    pooled0 = jnp.dot(S, x_ref[...], precision=_HI,
                      preferred_element_type=jnp.float32)
    pooled1 = jnp.dot(S, h, precision=_HI,
                      preferred_element_type=jnp.float32)
    o_ref[...] = (jnp.dot(pooled0, l0_ref[...], precision=_HI,
                          preferred_element_type=jnp.float32)
                  + jnp.dot(pooled1, l1_ref[...], precision=_HI,
                            preferred_element_type=jnp.float32)
                  + lb_ref[...])


def final_stage(z, Wu1, bu1, gscaled, bn_beta, batch, x, L0W, L1W, lb):
    nh = Wu1.shape[1]
    fs = lambda shp: pl.BlockSpec(shp, lambda: (0, 0))
    return pl.pallas_call(
        _final_kernel,
        in_specs=[fs((N, CH)), fs((CH, nh)), fs((1, nh)), fs((1, nh)),
                  fs((1, nh)), fs((1, N)), fs((N, D_IN)), fs((D_IN, OUT)),
                  fs((nh, OUT)), fs((1, OUT))],
        out_specs=fs((NB, OUT)),
        out_shape=jax.ShapeDtypeStruct((NB, OUT), jnp.float32),
    )(z, Wu1, bu1.reshape(1, nh), gscaled.reshape(1, nh),
      bn_beta.reshape(1, nh), batch.reshape(1, N), x, L0W, L1W,
      lb.reshape(1, OUT))


def kernel(x, edge_index, batch, W0, b0, W1, b1, W2, b2, p0, p1, Wu0, bu0,
           Wu1, bu1, bn_gamma, bn_beta, L0W, L0b, L1W, L1b):
    se, de = edge_index[0], edge_index[1]

    xw0 = mm_nn(x, W0)
    partself, partin = _edge_stats_sc(se, de)
    dis0, q0, u1 = norm0(partself, partin, xw0)
    tacc = _spmv_sc(se, de, u1)
    t1a = tacc[0:N, 0:CH]
    t1b = tacc[_ACC_ROWS:_ACC_ROWS + N, 0:CH]
    cur0, s1r, s1c = cur0_score(dis0, q0, t1a, t1b, xw0, b0, p0)

    posk1r, posk1c = topk_posk(s1r, s1c, K1)
    x1 = pool_gather(posk1r, cur0, s1c, K1)

    r0, ct0 = _build_rct_sc(se, de, posk1r.reshape(N))
    rb = addeye_cast(r0, posk1r)
    ctb = addeye_cast(ct0, posk1r)
    p_mat = mm_nt(rb, ctb)
    ap1, m2b = zerodiag(p_mat, add_eye=True)
    cs1 = degcol(ap1)

    v1 = mm_nn(x1, W1)
    cur1, s2r, s2c = gcn_dense(ap1, cs1, v1, b1, p=p1, relu=True)

    posk2r, posk2c = topk_posk(s2r, s2c, K2)
    x2 = pool_gather(posk2r, cur1, s2c, K2)

    o2 = onehot_rows(posk2r, K2)
    g2 = mm_nn(o2, m2b)
    h2 = mm_nt(m2b, o2)
    p2 = mm_nn(g2.astype(jnp.bfloat16), h2.astype(jnp.bfloat16))
    ap2, _ = zerodiag(p2, add_eye=False)
    cs2 = degcol(ap2)

    v2 = mm_nn(x2, W2)
    cur2 = gcn_dense(ap2, cs2, v2, b2, relu=True)

    mid = unpool_add(posk2c, cur2, cur1)
    vu0 = mm_nn(mid, Wu0)
    curu1 = gcn_dense(ap1, cs1, vu0, bu0, relu=True)

    full = unpool_add(posk1c, curu1, cur0)
    u2 = scale_rows(dis0, full)
    tacc2 = _spmv_sc(se, de, u2)
    z = scale_add(dis0, q0, tacc2[0:N, 0:CH],
                  tacc2[_ACC_ROWS:_ACC_ROWS + N, 0:CH], full)

    gscaled = bn_gamma / jnp.sqrt(1.0 + 1e-05)
    lb = L0b + L1b
    return final_stage(z, Wu1, bu1, gscaled, bn_beta, batch, x, L0W, L1W, lb)


# speedup vs baseline: 4.2331x; 1.0616x over previous
import functools
import jax
import jax.numpy as jnp
from jax import lax
from jax.experimental import pallas as pl
from jax.experimental.pallas import tpu as pltpu
from jax.experimental.pallas import tpu_sc as plsc

N = 4096
E = 65536
D_IN = 128
CH = 32
OUT = 16
NB = 64
K1 = 2048
K2 = 1024

_HI = lax.Precision.HIGHEST


def _mm_nn_kernel(prec, a_ref, b_ref, o_ref):
    o_ref[...] = jnp.dot(a_ref[...], b_ref[...], precision=prec,
                         preferred_element_type=jnp.float32)


def mm_nn(a, b, bm=512):
    prec = _HI if a.dtype == jnp.float32 else None
    M, K = a.shape
    _, Nn = b.shape
    bm = min(bm, M)
    return pl.pallas_call(
        functools.partial(_mm_nn_kernel, prec),
        grid=(M // bm,),
        in_specs=[pl.BlockSpec((bm, K), lambda i: (i, 0)),
                  pl.BlockSpec((K, Nn), lambda i: (0, 0))],
        out_specs=pl.BlockSpec((bm, Nn), lambda i: (i, 0)),
        out_shape=jax.ShapeDtypeStruct((M, Nn), jnp.float32),
    )(a, b)


def _mm_nt_kernel(a_ref, b_ref, o_ref):
    o_ref[...] = lax.dot_general(
        a_ref[...], b_ref[...], (((1,), (1,)), ((), ())),
        preferred_element_type=jnp.float32)


def mm_nt(a, b, bm=512, bn=512):
    M, K = a.shape
    Nn, _ = b.shape
    bm, bn = min(bm, M), min(bn, Nn)
    return pl.pallas_call(
        _mm_nt_kernel,
        grid=(M // bm, Nn // bn),
        in_specs=[pl.BlockSpec((bm, K), lambda i, j: (i, 0)),
                  pl.BlockSpec((bn, K), lambda i, j: (j, 0))],
        out_specs=pl.BlockSpec((bm, bn), lambda i, j: (i, j)),
        out_shape=jax.ShapeDtypeStruct((M, Nn), jnp.float32),
    )(a, b)


def _sc_mesh():
    return plsc.VectorSubcoreMesh(core_axis_name="c", subcore_axis_name="s")


_SC_PARAMS = pltpu.CompilerParams(needs_layout_passes=False)


_EPW = E // 32
_ACC_ROWS = 4224


def _edge_stats_sc(se, de):
    @functools.partial(
        pl.kernel,
        out_type=[jax.ShapeDtypeStruct((32, N), jnp.float32),
                  jax.ShapeDtypeStruct((32, N), jnp.float32)],
        mesh=_sc_mesh(),
        compiler_params=_SC_PARAMS,
        scratch_types=[pltpu.VMEM((_EPW,), jnp.int32),
                       pltpu.VMEM((_EPW,), jnp.int32),
                       pltpu.VMEM((N,), jnp.float32),
                       pltpu.VMEM((N,), jnp.float32),
                       pltpu.SemaphoreType.DMA],
    )
    def body(se_hbm, de_hbm, oself_hbm, oin_hbm, se_v, de_v, accs_v, acci_v,
             sem):
        c = lax.axis_index("c")
        s = lax.axis_index("s")
        wid = s * 2 + c
        base = wid * _EPW

        @pl.loop(0, N, step=16)
        def _(i):
            z = jnp.zeros((16,), jnp.float32)
            accs_v[pl.ds(i, 16)] = z
            acci_v[pl.ds(i, 16)] = z

        pltpu.sync_copy(se_hbm.at[pl.ds(base, _EPW)], se_v)
        pltpu.sync_copy(de_hbm.at[pl.ds(base, _EPW)], de_v)
        ones = jnp.ones((16,), jnp.float32)

        @pl.loop(0, _EPW, step=16)
        def _(j):
            sv = se_v[pl.ds(j, 16)]
            dv = de_v[pl.ds(j, 16)]
            m_self = sv == dv
            plsc.addupdate_scatter(accs_v, [dv], ones, mask=m_self)
            plsc.addupdate_scatter(acci_v, [dv], ones,
                                   mask=jnp.logical_not(m_self))

        pltpu.sync_copy(accs_v, oself_hbm.at[wid])
        pltpu.sync_copy(acci_v, oin_hbm.at[wid])

    return body(se, de)


def _spmv_sc(se, de, u):
    @functools.partial(
        pl.kernel,
        out_type=jax.ShapeDtypeStruct((2 * _ACC_ROWS, 128), jnp.float32),
        mesh=_sc_mesh(),
        compiler_params=_SC_PARAMS,
        scratch_types=[pltpu.VMEM((_EPW,), jnp.int32),
                       pltpu.VMEM((_EPW,), jnp.int32),
                       pltpu.VMEM((16, 128), jnp.int32),
                       pltpu.VMEM((128, 128), jnp.float32),
                       pltpu.VMEM((264, 128), jnp.float32),
                       pltpu.VMEM_SHARED((_ACC_ROWS, 128), jnp.float32),
                       pltpu.SemaphoreType.DMA],
    )
    def body(se_hbm, de_hbm, u_hbm, out_hbm, se_v, de_v, didx_v, rows_v,
             zbuf_v, acc_sh, sem):
        c = lax.axis_index("c")
        s = lax.axis_index("s")
        wid = s * 2 + c
        base = wid * _EPW

        @pl.loop(0, 264, step=1)
        def _(r):
            @pl.loop(0, 128, step=16)
            def _(cc):
                zbuf_v[r, pl.ds(cc, 16)] = jnp.zeros((16,), jnp.float32)

        pltpu.sync_copy(zbuf_v, acc_sh.at[pl.ds(s * 264, 264)])
        plsc.subcore_barrier()

        pltpu.sync_copy(se_hbm.at[pl.ds(base, _EPW)], se_v)
        pltpu.sync_copy(de_hbm.at[pl.ds(base, _EPW)], de_v)

        dummy = jnp.full((16,), 4096, jnp.int32) + s

        @pl.loop(0, _EPW, step=16)
        def _(j):
            sv = se_v[pl.ds(j, 16)]
            dv = de_v[pl.ds(j, 16)]
            dd = jnp.where(sv == dv, dummy, dv)
            didx_v[j // 128, pl.ds(j % 128, 16)] = dd

        @pl.loop(0, 16, step=1)
        def _(k):
            pltpu.async_copy(u_hbm.at[se_v.at[pl.ds(k * 128, 128)]],
                             rows_v, sem).wait()
            pltpu.sync_copy(rows_v, acc_sh.at[didx_v.at[k]], add=True)

        plsc.subcore_barrier()
        pltpu.sync_copy(acc_sh.at[pl.ds(s * 264, 264)],
                        out_hbm.at[pl.ds(c * _ACC_ROWS + s * 264, 264)])

    return body(se, de, u)


def _build_rct_sc(se, de, posk1):
    nw = N // 4

    @functools.partial(
        pl.kernel,
        out_type=[jax.ShapeDtypeStruct((K1, nw), jnp.int32),
                  jax.ShapeDtypeStruct((K1, nw), jnp.int32)],
        mesh=_sc_mesh(),
        compiler_params=_SC_PARAMS,
        scratch_types=[pltpu.VMEM((8192,), jnp.int32),
                       pltpu.VMEM((8192,), jnp.int32),
                       pltpu.VMEM((N,), jnp.int32),
                       pltpu.VMEM((64, nw), jnp.int32),
                       pltpu.SemaphoreType.DMA],
    )
    def body(se_hbm, de_hbm, posk_hbm, r0_hbm, ct0_hbm, se_v, de_v, posk_v,
             chunk_v, sem):
        c = lax.axis_index("c")
        s = lax.axis_index("s")
        wid = s * 2 + c
        row0 = wid * 64
        pltpu.sync_copy(posk_hbm, posk_v)

        one_i = jnp.ones((16,), jnp.int32)
        i16 = lax.iota(jnp.int32, 16)
        zero16 = jnp.zeros((16,), jnp.int32)
        c127 = jnp.full((16,), 1023, jnp.int32)
        c3 = jnp.full((16,), 3, jnp.int32)
        c8 = jnp.full((16,), 8, jnp.int32)

        def one_matrix(out_hbm, use_src_for_row):
            @pl.loop(0, 64, step=1)
            def _(r):
                @pl.loop(0, nw, step=16)
                def _(cc):
                    chunk_v[r, pl.ds(cc, 16)] = zero16

            @pl.loop(0, E, step=8192)
            def _(eb):
                pltpu.sync_copy(se_hbm.at[pl.ds(eb, 8192)], se_v)
                pltpu.sync_copy(de_hbm.at[pl.ds(eb, 8192)], de_v)

                @pl.loop(0, 8192, step=16)
                def _(j):
                    sv = se_v[pl.ds(j, 16)]
                    dv = de_v[pl.ds(j, 16)]
                    if use_src_for_row:
                        rk = plsc.load_gather(posk_v, [sv])
                        col = dv
                    else:
                        rk = plsc.load_gather(posk_v, [dv])
                        col = sv
                    rloc = rk - row0
                    valid = jnp.logical_and(
                        jnp.logical_and(rloc >= 0, rloc < 64),
                        sv != dv)
                    rsafe = jnp.where(valid, rloc, zero16)
                    word = jnp.bitwise_and(
                        lax.shift_right_logical(col, 2), c127)
                    bval = lax.shift_left(
                        one_i, jnp.bitwise_and(col, c3) * c8)
                    plsc.addupdate_scatter(chunk_v, [rsafe, word], bval,
                                           mask=valid)

            pltpu.sync_copy(chunk_v, out_hbm.at[pl.ds(row0, 64)])

        one_matrix(r0_hbm, True)
        one_matrix(ct0_hbm, False)

    r0w, ct0w = body(se, de, posk1)
    r0 = lax.bitcast_convert_type(r0w, jnp.int8).reshape(K1, N)
    ct0 = lax.bitcast_convert_type(ct0w, jnp.int8).reshape(K1, N)
    return r0, ct0


def _norm0_kernel(sc_ref, id_ref, xw_ref, dis_ref, q_ref, u_ref):
    ones = jnp.ones((32, 1), jnp.float32)
    selfcnt = lax.dot_general(sc_ref[...], ones, (((0,), (0,)), ((), ())),
                              precision=_HI,
                              preferred_element_type=jnp.float32)
    indeg = lax.dot_general(id_ref[...], ones, (((0,), (0,)), ((), ())),
                            precision=_HI,
                            preferred_element_type=jnp.float32)
    newd = jnp.where(selfcnt == 0.0, 2.0, selfcnt)
    deg = indeg + newd
    dis = jnp.where(deg > 0.0, lax.rsqrt(deg), 0.0)
    dis_ref[...] = dis
    q_ref[...] = dis * dis * newd
    u_ref[...] = jnp.concatenate(
        [dis * xw_ref[...], jnp.zeros((N, 128 - CH), jnp.float32)], axis=1)


def norm0(partself, partin, xw):
    col = pl.BlockSpec((N, 1), lambda: (0, 0))
    mat = pl.BlockSpec((N, CH), lambda: (0, 0))
    part = pl.BlockSpec((32, N), lambda: (0, 0))
    return pl.pallas_call(
        _norm0_kernel,
        in_specs=[part, part, mat],
        out_specs=[col, col, pl.BlockSpec((N, 128), lambda: (0, 0))],
        out_shape=[jax.ShapeDtypeStruct((N, 1), jnp.float32),
                   jax.ShapeDtypeStruct((N, 1), jnp.float32),
                   jax.ShapeDtypeStruct((N, 128), jnp.float32)],
    )(partself, partin, xw)


def _cur0_kernel(dis_ref, q_ref, t0_ref, t1_ref, xw_ref, b_ref, p_ref,
                 cur_ref, sr_ref, sc_ref):
    t = t0_ref[...] + t1_ref[...]
    cur = jnp.maximum(
        dis_ref[...] * t + q_ref[...] * xw_ref[...] + b_ref[...],
        0.0)
    cur_ref[...] = cur
    p = p_ref[...]
    pn = p / jnp.sqrt(jnp.sum(p * p))
    sr_ref[...] = jnp.tanh(lax.dot_general(
        pn, cur, (((1,), (1,)), ((), ())), precision=_HI,
        preferred_element_type=jnp.float32))
    sc_ref[...] = jnp.tanh(lax.dot_general(
        cur, pn, (((1,), (1,)), ((), ())), precision=_HI,
        preferred_element_type=jnp.float32))


def cur0_score(dis, q, t0, t1, xw, b, p):
    col = pl.BlockSpec((N, 1), lambda: (0, 0))
    mat = pl.BlockSpec((N, CH), lambda: (0, 0))
    vec = pl.BlockSpec((1, CH), lambda: (0, 0))
    return pl.pallas_call(
        _cur0_kernel,
        in_specs=[col, col, mat, mat, mat, vec, vec],
        out_specs=[mat, pl.BlockSpec((1, N), lambda: (0, 0)), col],
        out_shape=[jax.ShapeDtypeStruct((N, CH), jnp.float32),
                   jax.ShapeDtypeStruct((1, N), jnp.float32),
                   jax.ShapeDtypeStruct((N, 1), jnp.float32)],
    )(dis, q, t0, t1, xw, b.reshape(1, CH), p.reshape(1, CH))


def _topk_kernel(k, bm, sr_ref, sc_ref, srb_ref, scb_ref, pr_ref, pc_ref):
    i = pl.program_id(0)
    n = sr_ref.shape[1]
    s_row = sr_ref[...]
    s_col = sc_ref[...]
    s_row_blk = srb_ref[...].reshape(1, bm)
    idx_blk_l = lax.broadcasted_iota(jnp.int32, (1, bm), 1) + i * bm
    idx_col = lax.broadcasted_iota(jnp.int32, (n, 1), 0)
    gt = (s_col > s_row_blk).astype(jnp.float32)
    tie = jnp.logical_and(s_col == s_row_blk, idx_col < idx_blk_l)
    rank_l = jnp.sum(gt + tie.astype(jnp.float32), axis=0,
                     keepdims=True)
    pr_ref[...] = jnp.where(rank_l < k, rank_l.astype(jnp.int32),
                            -1).reshape(1, 1, bm)
    s_col_blk = scb_ref[...]
    idx_blk_c = lax.broadcasted_iota(jnp.int32, (bm, 1), 0) + i * bm
    idx_row = lax.broadcasted_iota(jnp.int32, (1, n), 1)
    gt2 = (s_row > s_col_blk).astype(jnp.float32)
    tie2 = jnp.logical_and(s_row == s_col_blk, idx_row < idx_blk_c)
    rank_c = jnp.sum(gt2 + tie2.astype(jnp.float32), axis=1,
                     keepdims=True)
    pc_ref[...] = jnp.where(rank_c < k, rank_c.astype(jnp.int32), -1)


def topk_posk(s_row, s_col, k, bm=512):
    n = s_row.shape[1]
    s_row3 = s_row.reshape(n // bm, 1, bm)
    pr3, pc = pl.pallas_call(
        functools.partial(_topk_kernel, k, bm),
        grid=(n // bm,),
        in_specs=[pl.BlockSpec((1, n), lambda i: (0, 0)),
                  pl.BlockSpec((n, 1), lambda i: (0, 0)),
                  pl.BlockSpec((1, 1, bm), lambda i: (i, 0, 0)),
                  pl.BlockSpec((bm, 1), lambda i: (i, 0))],
        out_specs=[pl.BlockSpec((1, 1, bm), lambda i: (i, 0, 0)),
                   pl.BlockSpec((bm, 1), lambda i: (i, 0))],
        out_shape=[jax.ShapeDtypeStruct((n // bm, 1, bm), jnp.int32),
                   jax.ShapeDtypeStruct((n, 1), jnp.int32)],
    )(s_row, s_col, s_row3, s_col)
    return pr3.reshape(1, n), pc


def _gather_kernel(bm, posk_ref, feat_ref, s_ref, o_ref):
    i = pl.program_id(0)
    posk = posk_ref[...]
    rr = lax.broadcasted_iota(jnp.int32, (bm, 1), 0) + i * bm
    oh = (posk == rr).astype(jnp.float32)
    fs = feat_ref[...] * s_ref[...]
    o_ref[...] = jnp.dot(oh, fs, precision=_HI,
                         preferred_element_type=jnp.float32)


def pool_gather(posk_row, feat, s_col, k, bm=512):
    n, f = feat.shape
    return pl.pallas_call(
        functools.partial(_gather_kernel, bm),
        grid=(k // bm,),
        in_specs=[pl.BlockSpec((1, n), lambda i: (0, 0)),
                  pl.BlockSpec((n, f), lambda i: (0, 0)),
                  pl.BlockSpec((n, 1), lambda i: (0, 0))],
        out_specs=pl.BlockSpec((bm, f), lambda i: (i, 0)),
        out_shape=jax.ShapeDtypeStruct((k, f), jnp.float32),
    )(posk_row, feat, s_col)


def _unpool_kernel(bm, posk_ref, cur_ref, res_ref, o_ref):
    posk_blk = posk_ref[...]
    k = cur_ref.shape[0]
    cc = lax.broadcasted_iota(jnp.int32, (1, k), 1)
    oh = (posk_blk == cc).astype(jnp.float32)
    up = jnp.dot(oh, cur_ref[...], precision=_HI,
                 preferred_element_type=jnp.float32)
    o_ref[...] = res_ref[...] + up


def unpool_add(posk_col, cur, res, bm=512):
    n, f = res.shape
    k = cur.shape[0]
    return pl.pallas_call(
        functools.partial(_unpool_kernel, bm),
        grid=(n // bm,),
        in_specs=[pl.BlockSpec((bm, 1), lambda i: (i, 0)),
                  pl.BlockSpec((k, f), lambda i: (0, 0)),
                  pl.BlockSpec((bm, f), lambda i: (i, 0))],
        out_specs=pl.BlockSpec((bm, f), lambda i: (i, 0)),
        out_shape=jax.ShapeDtypeStruct((n, f), jnp.float32),
    )(posk_col, cur, res)


def _addeye_cast_kernel(bm, m_ref, posk_ref, o_ref):
    i = pl.program_id(0)
    posk = posk_ref[...]
    rr = lax.broadcasted_iota(jnp.int32, (bm, 1), 0) + i * bm
    oh = (posk == rr).astype(jnp.float32)
    o_ref[...] = (m_ref[...].astype(jnp.float32) + oh).astype(jnp.bfloat16)


def addeye_cast(m, posk_row, bm=512):
    k, n = m.shape
    return pl.pallas_call(
        functools.partial(_addeye_cast_kernel, bm),
        grid=(k // bm,),
        in_specs=[pl.BlockSpec((bm, n), lambda i: (i, 0)),
                  pl.BlockSpec((1, n), lambda i: (0, 0))],
        out_specs=pl.BlockSpec((bm, n), lambda i: (i, 0)),
        out_shape=jax.ShapeDtypeStruct((k, n), jnp.bfloat16),
    )(m, posk_row)


def _onehot_rows_kernel(bm, posk_ref, o_ref):
    i = pl.program_id(0)
    posk = posk_ref[...]
    rr = lax.broadcasted_iota(jnp.int32, (bm, 1), 0) + i * bm
    o_ref[...] = (posk == rr).astype(jnp.bfloat16)


def onehot_rows(posk_row, k, bm=512):
    n = posk_row.shape[1]
    return pl.pallas_call(
        functools.partial(_onehot_rows_kernel, bm),
        grid=(k // bm,),
        in_specs=[pl.BlockSpec((1, n), lambda i: (0, 0))],
        out_specs=pl.BlockSpec((bm, n), lambda i: (i, 0)),
        out_shape=jax.ShapeDtypeStruct((k, n), jnp.bfloat16),
    )(posk_row)


def _zerodiag_cast_kernel(bm, add_eye, p_ref, o_ref, ob_ref):
    i = pl.program_id(0)
    j = pl.program_id(1)
    bn = p_ref.shape[1]
    rr = lax.broadcasted_iota(jnp.int32, (bm, 1), 0) + i * bm
    cc = lax.broadcasted_iota(jnp.int32, (1, bn), 1) + j * bn
    diag = (rr == cc).astype(jnp.float32)
    v = p_ref[...] * (1.0 - diag)
    o_ref[...] = v
    if add_eye:
        ob_ref[...] = (v + diag).astype(jnp.bfloat16)
    else:
        ob_ref[...] = v.astype(jnp.bfloat16)


def zerodiag(p, add_eye, bm=512):
    k = p.shape[0]
    return pl.pallas_call(
        functools.partial(_zerodiag_cast_kernel, bm, add_eye),
        grid=(k // bm, k // bm),
        in_specs=[pl.BlockSpec((bm, bm), lambda i, j: (i, j))],
        out_specs=[pl.BlockSpec((bm, bm), lambda i, j: (i, j)),
                   pl.BlockSpec((bm, bm), lambda i, j: (i, j))],
        out_shape=[jax.ShapeDtypeStruct((k, k), jnp.float32),
                   jax.ShapeDtypeStruct((k, k), jnp.bfloat16)],
    )(p)


def _degcol_kernel(ap_ref, o_ref):
    k = ap_ref.shape[0]
    ones = jnp.ones((k, 1), jnp.float32)
    o_ref[...] = lax.dot_general(ap_ref[...], ones, (((0,), (0,)), ((), ())),
                                 precision=_HI,
                                 preferred_element_type=jnp.float32)


def degcol(ap, bm=512):
    k = ap.shape[0]
    return pl.pallas_call(
        _degcol_kernel,
        grid=(k // bm,),
        in_specs=[pl.BlockSpec((k, bm), lambda i: (0, i))],
        out_specs=pl.BlockSpec((bm, 1), lambda i: (i, 0)),
        out_shape=jax.ShapeDtypeStruct((k, 1), jnp.float32),
    )(ap)


def _gcn_kernel(bm, relu, score, ap_ref, cs_ref, v_ref, csb_ref, vb_ref,
                b_ref, p_ref, o_ref, sr_ref, sc_ref):
    deg = cs_ref[...] + 2.0
    dis = jnp.where(deg > 0.0, lax.rsqrt(deg), 0.0)
    w = dis * v_ref[...]
    t_blk = lax.dot_general(ap_ref[...], w, (((0,), (0,)), ((), ())),
                            precision=_HI,
                            preferred_element_type=jnp.float32)
    degb = csb_ref[...] + 2.0
    dis_blk = jnp.where(degb > 0.0, lax.rsqrt(degb), 0.0)
    w_blk = dis_blk * vb_ref[...]
    o = dis_blk * (t_blk + 2.0 * w_blk) + b_ref[...]
    if relu:
        o = jnp.maximum(o, 0.0)
    o_ref[...] = o
    if score:
        p = p_ref[...]
        pn = p / jnp.sqrt(jnp.sum(p * p))
        sr_ref[...] = jnp.tanh(lax.dot_general(
            pn, o, (((1,), (1,)), ((), ())), precision=_HI,
            preferred_element_type=jnp.float32)).reshape(1, 1, bm)
        sc_ref[...] = jnp.tanh(lax.dot_general(
            o, pn, (((1,), (1,)), ((), ())), precision=_HI,
            preferred_element_type=jnp.float32))


def gcn_dense(ap, cs_col, v, b, p=None, relu=True, bm=512):
    k, f = v.shape
    score = p is not None
    if p is None:
        p = jnp.zeros((CH,), jnp.float32)
    outs = pl.pallas_call(
        functools.partial(_gcn_kernel, bm, relu, score),
        grid=(k // bm,),
        in_specs=[pl.BlockSpec((k, bm), lambda i: (0, i)),
                  pl.BlockSpec((k, 1), lambda i: (0, 0)),
                  pl.BlockSpec((k, f), lambda i: (0, 0)),
                  pl.BlockSpec((bm, 1), lambda i: (i, 0)),
                  pl.BlockSpec((bm, f), lambda i: (i, 0)),
                  pl.BlockSpec((1, f), lambda i: (0, 0)),
                  pl.BlockSpec((1, CH), lambda i: (0, 0))],
        out_specs=[pl.BlockSpec((bm, f), lambda i: (i, 0)),
                   pl.BlockSpec((1, 1, bm), lambda i: (i, 0, 0)),
                   pl.BlockSpec((bm, 1), lambda i: (i, 0))],
        out_shape=[jax.ShapeDtypeStruct((k, f), jnp.float32),
                   jax.ShapeDtypeStruct((k // bm, 1, bm), jnp.float32),
                   jax.ShapeDtypeStruct((k, 1), jnp.float32)],
    )(ap, cs_col, v, cs_col, v, b.reshape(1, f), p.reshape(1, CH))
    if score:
        return outs[0], outs[1].reshape(1, k), outs[2]
    return outs[0]


def _scale_kernel(dis_ref, c_ref, u_ref):
    n, f = c_ref.shape
    u_ref[...] = jnp.concatenate(
        [dis_ref[...] * c_ref[...], jnp.zeros((n, 128 - f), jnp.float32)],
        axis=1)


def scale_rows(dis_col, c):
    n, f = c.shape
    return pl.pallas_call(
        _scale_kernel,
        in_specs=[pl.BlockSpec((n, 1), lambda: (0, 0)),
                  pl.BlockSpec((n, f), lambda: (0, 0))],
        out_specs=pl.BlockSpec((n, 128), lambda: (0, 0)),
        out_shape=jax.ShapeDtypeStruct((n, 128), jnp.float32),
    )(dis_col, c)


def _scale_add_kernel(dis_ref, q_ref, t0_ref, t1_ref, c_ref, z_ref):
    t = t0_ref[...] + t1_ref[...]
    z_ref[...] = dis_ref[...] * t + q_ref[...] * c_ref[...]


def scale_add(dis_col, q_col, t0, t1, c):
    n, f = c.shape
    col = pl.BlockSpec((n, 1), lambda: (0, 0))
    mat = pl.BlockSpec((n, f), lambda: (0, 0))
    return pl.pallas_call(
        _scale_add_kernel,
        in_specs=[col, col, mat, mat, mat],
        out_specs=mat,
        out_shape=jax.ShapeDtypeStruct((n, f), jnp.float32),
    )(dis_col, q_col, t0, t1, c)


def _final_kernel(z_ref, wu_ref, bu_ref, g_ref, be_ref, batch_ref, x_ref,
                  l0_ref, l1_ref, lb_ref, o_ref):
    h = jnp.dot(z_ref[...], wu_ref[...], precision=_HI,
                preferred_element_type=jnp.float32) + bu_ref[...]
    h = h * g_ref[...] + be_ref[...]
    h = jnp.maximum(h, 0.0)
    batch = batch_ref[...]
    bb = lax.broadcasted_iota(jnp.int32, (NB, 1), 0)
    S = (batch == bb).astype(jnp.float32)
    pooled0 = jnp.dot(S, x_ref[...], precision=_HI,
                      preferred_element_type=jnp.float32)
    pooled1 = jnp.dot(S, h, precision=_HI,
                      preferred_element_type=jnp.float32)
    o_ref[...] = (jnp.dot(pooled0, l0_ref[...], precision=_HI,
                          preferred_element_type=jnp.float32)
                  + jnp.dot(pooled1, l1_ref[...], precision=_HI,
                            preferred_element_type=jnp.float32)
                  + lb_ref[...])


def final_stage(z, Wu1, bu1, gscaled, bn_beta, batch, x, L0W, L1W, lb):
    nh = Wu1.shape[1]
    fs = lambda shp: pl.BlockSpec(shp, lambda: (0, 0))
    return pl.pallas_call(
        _final_kernel,
        in_specs=[fs((N, CH)), fs((CH, nh)), fs((1, nh)), fs((1, nh)),
                  fs((1, nh)), fs((1, N)), fs((N, D_IN)), fs((D_IN, OUT)),
                  fs((nh, OUT)), fs((1, OUT))],
        out_specs=fs((NB, OUT)),
        out_shape=jax.ShapeDtypeStruct((NB, OUT), jnp.float32),
    )(z, Wu1, bu1.reshape(1, nh), gscaled.reshape(1, nh),
      bn_beta.reshape(1, nh), batch.reshape(1, N), x, L0W, L1W,
      lb.reshape(1, OUT))


def kernel(x, edge_index, batch, W0, b0, W1, b1, W2, b2, p0, p1, Wu0, bu0,
           Wu1, bu1, bn_gamma, bn_beta, L0W, L0b, L1W, L1b):
    se, de = edge_index[0], edge_index[1]

    xw0 = mm_nn(x, W0)
    partself, partin = _edge_stats_sc(se, de)
    dis0, q0, u1 = norm0(partself, partin, xw0)
    tacc = _spmv_sc(se, de, u1)
    t1a = tacc[0:N, 0:CH]
    t1b = tacc[_ACC_ROWS:_ACC_ROWS + N, 0:CH]
    cur0, s1r, s1c = cur0_score(dis0, q0, t1a, t1b, xw0, b0, p0)

    posk1r, posk1c = topk_posk(s1r, s1c, K1)
    x1 = pool_gather(posk1r, cur0, s1c, K1)

    r0, ct0 = _build_rct_sc(se, de, posk1r.reshape(N))
    rb = addeye_cast(r0, posk1r)
    ctb = addeye_cast(ct0, posk1r)
    p_mat = mm_nt(rb, ctb)
    ap1, m2b = zerodiag(p_mat, add_eye=True)
    cs1 = degcol(ap1)

    v1 = mm_nn(x1, W1)
    cur1, s2r, s2c = gcn_dense(ap1, cs1, v1, b1, p=p1, relu=True)

    posk2r, posk2c = topk_posk(s2r, s2c, K2)
    x2 = pool_gather(posk2r, cur1, s2c, K2)

    o2 = onehot_rows(posk2r, K2)
    g2 = mm_nn(o2, m2b)
    h2 = mm_nt(m2b, o2)
    p2 = mm_nn(g2.astype(jnp.bfloat16), h2.astype(jnp.bfloat16))
    ap2, _ = zerodiag(p2, add_eye=False)
    cs2 = degcol(ap2)

    v2 = mm_nn(x2, W2)
    cur2 = gcn_dense(ap2, cs2, v2, b2, relu=True)

    mid = unpool_add(posk2c, cur2, cur1)
    vu0 = mm_nn(mid, Wu0)
    curu1 = gcn_dense(ap1, cs1, vu0, bu0, relu=True)

    full = unpool_add(posk1c, curu1, cur0)
    u2 = scale_rows(dis0, full)
    tacc2 = _spmv_sc(se, de, u2)
    z = scale_add(dis0, q0, tacc2[0:N, 0:CH],
                  tacc2[_ACC_ROWS:_ACC_ROWS + N, 0:CH], full)

    gscaled = bn_gamma / jnp.sqrt(1.0 + 1e-05)
    lb = L0b + L1b
    return final_stage(z, Wu1, bu1, gscaled, bn_beta, batch, x, L0W, L1W, lb)
